# Initial kernel scaffold; baseline (speedup 1.0000x reference)
import jax
import jax.numpy as jnp
from jax.experimental import pallas as pl


def kernel(audio_feat, cls_tok, Wq, Wk, Wv, Wo, W1, b1, W2, b2, ln1_g, ln1_b, ln2_g, ln2_b, proj_W, proj_b, bn_scale, bn_bias, token_emb, audio_len):
    raise NotImplementedError("write your pallas kernel here")


# speedup vs baseline: 4.0700x; 4.0700x over previous
import functools

import jax
import jax.numpy as jnp
from jax import lax
from jax.experimental import pallas as pl
from jax.experimental.pallas import tpu as pltpu
from jax.experimental.pallas import tpu_sc as plsc

B, T, D = 32, 512, 768
KW = 8
HEADS, DH = 12, 64
FF = 3072
TD = 512
VOCAB = 49408
NKW = B * KW


def _attn_body(len_ref, af_ref, cls_ref, wq_ref, wk_ref, wv_ref, out_ref,
               q_s, kc_s, vc_s):
    b = pl.program_id(0)

    @pl.when(b == 0)
    def _():
        cls = cls_ref[...]
        q_s[...] = jnp.dot(cls, wq_ref[...], preferred_element_type=jnp.float32)
        kc_s[...] = jnp.dot(cls, wk_ref[...], preferred_element_type=jnp.float32)
        vc_s[...] = jnp.dot(cls, wv_ref[...], preferred_element_type=jnp.float32)

    audio = af_ref[0]
    ka = jnp.dot(audio, wk_ref[...], preferred_element_type=jnp.float32)
    va = jnp.dot(audio, wv_ref[...], preferred_element_type=jnp.float32)
    kf = jnp.concatenate([kc_s[...], ka], axis=0)
    vf = jnp.concatenate([vc_s[...], va], axis=0)

    alen = len_ref[b]
    col = lax.broadcasted_iota(jnp.int32, (1, KW + T), 1)
    pad = col >= alen + KW

    scale = jnp.float32(1.0 / 8.0)
    neg = jnp.float32(-1e9)

    for h in range(HEADS):
        sl = slice(h * DH, (h + 1) * DH)
        q_h = q_s[:, sl] * scale
        s = lax.dot_general(q_h, kf[:, sl],
                            (((1,), (1,)), ((), ())),
                            preferred_element_type=jnp.float32)
        s = jnp.where(pad, neg, s)
        m = jnp.max(s, axis=1, keepdims=True)
        e = jnp.exp(s - m)
        l = jnp.sum(e, axis=1, keepdims=True)
        out_ref[0, :, sl] = jnp.dot(e / l, vf[:, sl],
                                    preferred_element_type=jnp.float32)


def _attention(audio_feat, cls2, wq, wk, wv, audio_len):
    return pl.pallas_call(
        _attn_body,
        grid=(B,),
        in_specs=[
            pl.BlockSpec(memory_space=pltpu.SMEM),
            pl.BlockSpec((1, T, D), lambda b: (b, 0, 0)),
            pl.BlockSpec((KW, D), lambda b: (0, 0)),
            pl.BlockSpec((D, D), lambda b: (0, 0)),
            pl.BlockSpec((D, D), lambda b: (0, 0)),
            pl.BlockSpec((D, D), lambda b: (0, 0)),
        ],
        out_specs=pl.BlockSpec((1, KW, D), lambda b: (b, 0, 0)),
        out_shape=jax.ShapeDtypeStruct((B, KW, D), jnp.float32),
        scratch_shapes=[
            pltpu.VMEM((KW, D), jnp.float32),
            pltpu.VMEM((KW, D), jnp.float32),
            pltpu.VMEM((KW, D), jnp.float32),
        ],
    )(audio_len, audio_feat, cls2, wq, wk, wv)


def _ln(x, g, b):
    mu = jnp.mean(x, axis=-1, keepdims=True)
    var = jnp.mean(jnp.square(x - mu), axis=-1, keepdims=True)
    return (x - mu) / jnp.sqrt(var + 1e-5) * g + b


def _head_body(ctx_ref, res_ref, wo_ref, l1g_ref, l1b_ref, w1_ref, b1_ref,
               w2_ref, b2_ref, l2g_ref, l2b_ref, pw_ref, pb_ref, bs_ref,
               bb_ref, out_ref):
    x = res_ref[...] + jnp.dot(ctx_ref[...], wo_ref[...],
                               preferred_element_type=jnp.float32)
    x = _ln(x, l1g_ref[...], l1b_ref[...])
    h = jnp.maximum(
        jnp.dot(x, w1_ref[...], preferred_element_type=jnp.float32)
        + b1_ref[...], 0.0)
    y = jnp.dot(h, w2_ref[...], preferred_element_type=jnp.float32) + b2_ref[...]
    x = _ln(x + y, l2g_ref[...], l2b_ref[...])
    kw = jnp.dot(x, pw_ref[...], preferred_element_type=jnp.float32) + pb_ref[...]
    kw = kw * bs_ref[...] + bb_ref[...]
    nrm = jnp.sqrt(jnp.sum(jnp.square(kw), axis=1, keepdims=True))
    out_ref[...] = kw / (nrm + 1e-8)


def _head(ctx2, res, wo, l1g, l1b, w1, b1, w2, b2, l2g, l2b, pw, pb, bs, bb):
    vec = lambda v: v.reshape(1, -1)
    return pl.pallas_call(
        _head_body,
        out_shape=jax.ShapeDtypeStruct((NKW, TD), jnp.float32),
    )(ctx2, res, wo, vec(l1g), vec(l1b), w1, vec(b1), w2, vec(b2),
      vec(l2g), vec(l2b), pw, vec(pb), vec(bs), vec(bb))


VTILE = 2048
VGRID = (VOCAB + VTILE - 1) // VTILE


def _argmax_body(kw_ref, emb_ref, idx_ref, best_v, best_i):
    v = pl.program_id(0)
    emb = emb_ref[...]
    nrm = jnp.sqrt(jnp.sum(jnp.square(emb), axis=1, keepdims=True))
    emb_n = emb / (nrm + 1e-8)
    raw = lax.dot_general(kw_ref[...], emb_n, (((1,), (1,)), ((), ())),
                          preferred_element_type=jnp.float32)
    gcol = v * VTILE + lax.broadcasted_iota(jnp.int32, (1, VTILE), 1)
    cos = jnp.where(gcol < VOCAB, raw, jnp.float32(-3e38))
    tmax = jnp.max(cos, axis=1, keepdims=True)
    targ = jnp.min(jnp.where(cos == tmax, gcol, jnp.int32(2**31 - 1)),
                   axis=1, keepdims=True)

    @pl.when(v == 0)
    def _():
        best_v[...] = tmax
        best_i[...] = targ

    @pl.when(v > 0)
    def _():
        upd = tmax > best_v[...]
        best_v[...] = jnp.where(upd, tmax, best_v[...])
        best_i[...] = jnp.where(upd, targ, best_i[...])

    @pl.when(v == VGRID - 1)
    def _():
        idx_ref[...] = best_i[...]


def _vq_argmax(kw, emb):
    return pl.pallas_call(
        _argmax_body,
        grid=(VGRID,),
        in_specs=[
            pl.BlockSpec((NKW, TD), lambda v: (0, 0)),
            pl.BlockSpec((VTILE, TD), lambda v: (v, 0)),
        ],
        out_specs=pl.BlockSpec((NKW, 1), lambda v: (0, 0)),
        out_shape=jax.ShapeDtypeStruct((NKW, 1), jnp.int32),
        scratch_shapes=[
            pltpu.VMEM((NKW, 1), jnp.float32),
            pltpu.VMEM((NKW, 1), jnp.int32),
        ],
    )(kw, emb)


_NC, _NS = 2, 16
_NW = _NC * _NS
_BPW = NKW // _NW


def _sc_gather_body(emb_hbm, idx_hbm, out_hbm, idx_v, rows_v, sem):
    wid = lax.axis_index("s") * _NC + lax.axis_index("c")
    base = wid * _BPW
    pltpu.sync_copy(idx_hbm.at[pl.ds(base, _BPW)], idx_v)
    pltpu.async_copy(emb_hbm.at[idx_v], rows_v, sem).wait()
    pltpu.sync_copy(rows_v, out_hbm.at[pl.ds(base, _BPW)])


@functools.cache
def _sc_gather():
    return pl.kernel(
        _sc_gather_body,
        out_type=jax.ShapeDtypeStruct((NKW, TD), jnp.float32),
        mesh=plsc.VectorSubcoreMesh(core_axis_name="c", subcore_axis_name="s",
                                    num_cores=_NC, num_subcores=_NS),
        scratch_types=[
            pltpu.VMEM((_BPW,), jnp.int32),
            pltpu.VMEM((_BPW, TD), jnp.float32),
            pltpu.SemaphoreType.DMA,
        ],
    )


def kernel(audio_feat, cls_tok, Wq, Wk, Wv, Wo, W1, b1, W2, b2, ln1_g, ln1_b,
           ln2_g, ln2_b, proj_W, proj_b, bn_scale, bn_bias, token_emb,
           audio_len):
    cls2 = cls_tok.reshape(KW, D)
    ctx = _attention(audio_feat, cls2, Wq, Wk, Wv, audio_len)
    res = jnp.tile(cls2, (B, 1))
    kw = _head(ctx.reshape(NKW, D), res, Wo, ln1_g, ln1_b, W1, b1, W2, b2,
               ln2_g, ln2_b, proj_W, proj_b, bn_scale, bn_bias)
    idx = _vq_argmax(kw, token_emb).reshape(NKW)
    out = _sc_gather()(token_emb, idx)
    return out.reshape(B, KW, TD)
